# baseline (speedup 1.0000x reference)
import functools

import jax
import jax.numpy as jnp
from jax import lax
from jax.experimental import pallas as pl
from jax.experimental.pallas import tpu as pltpu
from jax.experimental.pallas import tpu_sc as plsc

BATCH = 16384
DIM = 64
NUM_CORES = 2
NUM_SUBCORES = 16
NW = NUM_CORES * NUM_SUBCORES
BPW = BATCH // NW


def _gather_body(ut_hbm, it_hbm, x0_hbm, x1_hbm, u_out, i_out,
                 idx_u, idx_i, rows_u, rows_i, sem_u, sem_i):
    wid = lax.axis_index("s") * NUM_CORES + lax.axis_index("c")
    base = wid * BPW
    pltpu.sync_copy(x0_hbm.at[pl.ds(base, BPW)], idx_u)
    pltpu.sync_copy(x1_hbm.at[pl.ds(base, BPW)], idx_i)
    cp_u = pltpu.async_copy(ut_hbm.at[idx_u], rows_u, sem_u)
    cp_i = pltpu.async_copy(it_hbm.at[idx_i], rows_i, sem_i)
    cp_u.wait()
    cp_i.wait()
    pltpu.sync_copy(rows_u, u_out.at[pl.ds(base, BPW)])
    pltpu.sync_copy(rows_i, i_out.at[pl.ds(base, BPW)])


def _sc_gather(ut, it, x0, x1):
    k = pl.kernel(
        _gather_body,
        out_type=(
            jax.ShapeDtypeStruct((BATCH, DIM), jnp.float32),
            jax.ShapeDtypeStruct((BATCH, DIM), jnp.float32),
        ),
        mesh=plsc.VectorSubcoreMesh(core_axis_name="c", subcore_axis_name="s"),
        scratch_types=[
            pltpu.VMEM((BPW,), jnp.int32),
            pltpu.VMEM((BPW,), jnp.int32),
            pltpu.VMEM((BPW, DIM), jnp.float32),
            pltpu.VMEM((BPW, DIM), jnp.float32),
            pltpu.SemaphoreType.DMA,
            pltpu.SemaphoreType.DMA,
        ],
        compiler_params=pltpu.CompilerParams(use_tc_tiling_on_sc=False),
    )
    return k(ut, it, x0, x1)


def _dense_body(u_ref, i_ref, wt_ref, o_ref):
    f = jnp.dot(u_ref[...], wt_ref[...], preferred_element_type=jnp.float32)
    s = jnp.sum(f * i_ref[...], axis=1)
    o_ref[...] = 1.0 / (1.0 + jnp.exp(-s))


def _dense(u, i, wt, blk=2048):
    grid = BATCH // blk
    return pl.pallas_call(
        _dense_body,
        grid=(grid,),
        in_specs=[
            pl.BlockSpec((blk, DIM), lambda j: (j, 0)),
            pl.BlockSpec((blk, DIM), lambda j: (j, 0)),
            pl.BlockSpec((DIM, DIM), lambda j: (0, 0)),
        ],
        out_specs=pl.BlockSpec((blk,), lambda j: (j,)),
        out_shape=jax.ShapeDtypeStruct((BATCH,), jnp.float32),
    )(u, i, wt)


def kernel(x, target_user_table, target_item_table, W_shared, W_target_only):
    x0 = x[:, 0].astype(jnp.int32)
    x1 = x[:, 1].astype(jnp.int32)
    wt = jnp.concatenate([W_shared, W_target_only], axis=0).T
    u_g, i_g = _sc_gather(target_user_table, target_item_table, x0, x1)
    return _dense(u_g, i_g, wt)
